# Initial kernel scaffold; baseline (speedup 1.0000x reference)
import jax
import jax.numpy as jnp
from jax.experimental import pallas as pl


def kernel(x, gate1, gate2, gate3, gate4, Wc, bc, Wp, bp):
    raise NotImplementedError("write your pallas kernel here")


# speedup vs baseline: 4.4815x; 4.4815x over previous
import jax
import jax.numpy as jnp
from jax.experimental import pallas as pl
from jax.experimental.pallas import tpu as pltpu

NUM_EXPERTS = 8
NUM_GATES = 4
B, H, W, C = 4, 32, 32, 128
PIX = H * W
EALL = NUM_EXPERTS * C


def _shift(v, oy, ox):
    if oy > 0:
        v = jnp.concatenate([v[oy:], jnp.zeros((oy, W, C), v.dtype)], axis=0)
    elif oy < 0:
        v = jnp.concatenate([jnp.zeros((-oy, W, C), v.dtype), v[:H + oy]], axis=0)
    if ox > 0:
        v = jnp.concatenate([v[:, ox:], jnp.zeros((H, ox, C), v.dtype)], axis=1)
    elif ox < 0:
        v = jnp.concatenate([jnp.zeros((H, -ox, C), v.dtype), v[:, :W + ox]], axis=1)
    return v


def _moe_kernel(x_ref, g_ref, wt_ref, bc_ref, wp_ref, bp_ref,
                y1_ref, y2_ref, y3_ref, y4_ref, loss_ref, usage_ref):
    i = pl.program_id(0)
    xv = x_ref[0]

    x2d = xv.reshape(PIX, C)
    x0 = jnp.sum(x2d, axis=0, keepdims=True) * (1.0 / PIX)
    logits = jnp.dot(x0, g_ref[...], preferred_element_type=jnp.float32)

    iota = jax.lax.broadcasted_iota(jnp.int32, (1, NUM_EXPERTS), 1)
    coeff_rows = []
    prob_rows = []
    for g in range(NUM_GATES):
        lg = logits[:, g * NUM_EXPERTS:(g + 1) * NUM_EXPERTS]
        lg = lg - jnp.max(lg, axis=1, keepdims=True)
        el = jnp.exp(lg)
        p = el / jnp.sum(el, axis=1, keepdims=True)
        prob_rows.append(p)
        m0 = jnp.max(p, axis=1, keepdims=True)
        i0 = jnp.min(jnp.where(p == m0, iota, NUM_EXPERTS), axis=1, keepdims=True)
        pm = jnp.where(iota == i0, -jnp.inf, p)
        m1 = jnp.max(pm, axis=1, keepdims=True)
        i1 = jnp.min(jnp.where(pm == m1, iota, NUM_EXPERTS), axis=1, keepdims=True)
        t = jnp.exp(m1 - m0)
        w0 = 1.0 / (1.0 + t)
        w1 = 1.0 - w0
        coeff_rows.append(w0 * (iota == i0).astype(jnp.float32)
                          + w1 * (iota == i1).astype(jnp.float32))
    probs = jnp.concatenate(prob_rows, axis=0)

    @pl.when(i == 0)
    def _():
        usage_ref[...] = probs

    @pl.when(i > 0)
    def _():
        usage_ref[...] += probs

    u = bc_ref[...] * jnp.ones((PIX, 1), jnp.float32)
    for t in range(9):
        oy, ox = t // 3 - 1, t % 3 - 1
        sx = _shift(xv, oy, ox).reshape(PIX, C)
        u = u + jnp.dot(sx, wt_ref[t], preferred_element_type=jnp.float32)

    accs = [jnp.zeros((PIX, C), jnp.float32) for _ in range(NUM_GATES)]
    for e in range(NUM_EXPERTS):
        ue = u[:, e * C:(e + 1) * C]
        sq = jnp.sum(ue * ue, axis=1, keepdims=True)
        f = sq / ((1.0 + sq) * (jnp.sqrt(sq) + 1e-8))
        se = ue * f
        ve = jnp.dot(se, wp_ref[e], preferred_element_type=jnp.float32) + bp_ref[e]
        for g in range(NUM_GATES):
            c = coeff_rows[g][:, e:e + 1]
            accs[g] = accs[g] + c * ve

    y1_ref[...] = accs[0][None]
    y2_ref[...] = accs[1][None]
    y3_ref[...] = accs[2][None]
    y4_ref[...] = accs[3][None]

    @pl.when(i == B - 1)
    def _():
        usage = usage_ref[...]
        mean = jnp.mean(usage, axis=1, keepdims=True)
        var = jnp.sum((usage - mean) ** 2, axis=1, keepdims=True) / (NUM_EXPERTS - 1)
        cv = var / (mean * mean + 1e-10)
        total = jnp.sum(cv, axis=0, keepdims=True)
        loss_ref[...] = jnp.broadcast_to(total, (1, NUM_EXPERTS))


def kernel(x, gate1, gate2, gate3, gate4, Wc, bc, Wp, bp):
    xt = jnp.transpose(x, (0, 2, 3, 1))
    gcat = jnp.concatenate([gate1, gate2, gate3, gate4], axis=1)
    wt = jnp.transpose(Wc, (3, 4, 2, 0, 1)).reshape(9, C, EALL)
    bc_all = bc.reshape(1, EALL)
    wpt = jnp.transpose(Wp[:, :, :, 0, 0], (0, 2, 1))
    bp3 = bp.reshape(NUM_EXPERTS, 1, C)

    grid = (B,)
    outs = pl.pallas_call(
        _moe_kernel,
        grid=grid,
        in_specs=[
            pl.BlockSpec((1, H, W, C), lambda i: (i, 0, 0, 0)),
            pl.BlockSpec((C, NUM_GATES * NUM_EXPERTS), lambda i: (0, 0)),
            pl.BlockSpec((9, C, EALL), lambda i: (0, 0, 0)),
            pl.BlockSpec((1, EALL), lambda i: (0, 0)),
            pl.BlockSpec((NUM_EXPERTS, C, C), lambda i: (0, 0, 0)),
            pl.BlockSpec((NUM_EXPERTS, 1, C), lambda i: (0, 0, 0)),
        ],
        out_specs=[
            pl.BlockSpec((1, PIX, C), lambda i: (i, 0, 0)),
            pl.BlockSpec((1, PIX, C), lambda i: (i, 0, 0)),
            pl.BlockSpec((1, PIX, C), lambda i: (i, 0, 0)),
            pl.BlockSpec((1, PIX, C), lambda i: (i, 0, 0)),
            pl.BlockSpec((1, NUM_EXPERTS), lambda i: (0, 0)),
        ],
        out_shape=[
            jax.ShapeDtypeStruct((B, PIX, C), jnp.float32),
            jax.ShapeDtypeStruct((B, PIX, C), jnp.float32),
            jax.ShapeDtypeStruct((B, PIX, C), jnp.float32),
            jax.ShapeDtypeStruct((B, PIX, C), jnp.float32),
            jax.ShapeDtypeStruct((1, NUM_EXPERTS), jnp.float32),
        ],
        scratch_shapes=[pltpu.VMEM((NUM_GATES, NUM_EXPERTS), jnp.float32)],
        compiler_params=pltpu.CompilerParams(
            dimension_semantics=("arbitrary",)),
    )(xt, gcat, wt, bc_all, wpt, bp3)

    ys = [o.reshape(B, H, W, C).transpose(0, 3, 1, 2) for o in outs[:4]]
    l = outs[4][0, 0].reshape(())
    return (ys[0], ys[1], ys[2], ys[3], l)
